# Initial kernel scaffold; baseline (speedup 1.0000x reference)
import jax
import jax.numpy as jnp
from jax.experimental import pallas as pl


def kernel(xs0, xs1, xs2, W0, W1, W2, bos0, bos1, bos2, eos0, eos1, eos2):
    raise NotImplementedError("write your pallas kernel here")


# speedup vs baseline: 5.1498x; 5.1498x over previous
import functools

import jax
import jax.numpy as jnp
from jax import lax
from jax.experimental import pallas as pl
from jax.experimental.pallas import tpu as pltpu
from jax.experimental.pallas import tpu_sc as plsc

B, L = 4096, 50
VOCAB = 100000
D = 64
NFIELD = 3
SEQ = L + 2
DTOT = NFIELD * D
NROWS = B * SEQ * NFIELD
ROWS_PER_BATCH = SEQ * NFIELD

NW = 32
BW = B // NW
TILE = 128
IDXROWS = B * L // TILE
ROWS_W = BW * L // TILE
K = 10
NCHUNK = ROWS_W // K

PADP = 2 * NFIELD
PTILE = 96
NPTILE = BW * PADP // PTILE

_mesh = plsc.VectorSubcoreMesh(core_axis_name="c", subcore_axis_name="s")


@functools.partial(
    pl.kernel,
    out_type=jax.ShapeDtypeStruct((NROWS, D), jnp.float32),
    mesh=_mesh,
    scratch_types=[
        pltpu.VMEM((56, TILE), jnp.int32),
        pltpu.VMEM((K, TILE), jnp.int32),
        pltpu.VMEM((K, TILE, D), jnp.float32),
        pltpu.VMEM((PTILE, D), jnp.float32),
        pltpu.VMEM((NPTILE, PTILE), jnp.int32),
        pltpu.SemaphoreType.DMA,
        pltpu.SemaphoreType.DMA,
        pltpu.SemaphoreType.DMA,
    ],
    compiler_params=pltpu.CompilerParams(use_tc_tiling_on_sc=False),
)
def _encode(xs0, xs1, xs2, w0, w1, w2, b0, b1, b2, e0, e1, e2, out,
            xsall, dstbuf, rowsbuf, padbuf, padidx, gsem, ssem, psem):
    w = lax.axis_index("s") * 2 + lax.axis_index("c")
    s_off = lax.rem(w * ROWS_W, 8)
    a0 = pl.multiple_of(w * ROWS_W - s_off, 8)
    iota = lax.iota(jnp.int32, 16)
    c6 = jnp.full((16,), 6, jnp.int32)
    cl = jnp.full((16,), L, jnp.int32)

    for r, src in enumerate((b0, b1, b2, e0, e1, e2)):
        pltpu.sync_copy(src, padbuf.at[r])
    for p in range(PADP, PTILE):
        for c4 in range(D // 16):
            padbuf[p, pl.ds(c4 * 16, 16)] = padbuf[p - PADP, pl.ds(c4 * 16, 16)]

    for j in range(NPTILE):
        for v in range(PTILE // 16):
            p = j * PTILE + v * 16 + iota
            q = lax.div(p, c6)
            r = p - 6 * q
            bat = w * BW + q
            dst = ROWS_PER_BATCH * bat + r + jnp.where(r < 3, 0, 150)
            padidx[j, pl.ds(v * 16, 16)] = dst
    pad_descs = [
        pltpu.async_copy(padbuf, out.at[padidx.at[j]], psem)
        for j in range(NPTILE)
    ]

    for i, (xs, tbl) in enumerate(((xs0, w0), (xs1, w1), (xs2, w2))):
        pltpu.sync_copy(xs.at[pl.ds(a0, 56)], xsall)

        def chunk(c, _, tbl=tbl, i=i):
            row0 = w * ROWS_W + c * K
            gets = [
                pltpu.async_copy(
                    tbl.at[xsall.at[s_off + c * K + b]], rowsbuf.at[b], gsem)
                for b in range(K)
            ]
            for b in range(K):
                for v in range(TILE // 16):
                    t = (row0 + b) * TILE + v * 16 + iota
                    q = lax.div(t, cl)
                    dstbuf[b, pl.ds(v * 16, 16)] = 3 * t + 6 * q + (3 + i)
            for g in gets:
                g.wait()
            puts = [
                pltpu.async_copy(rowsbuf.at[b], out.at[dstbuf.at[b]], ssem)
                for b in range(K)
            ]
            for s in puts:
                s.wait()
            return 0

        lax.fori_loop(0, NCHUNK, chunk, 0)

    for pd in pad_descs:
        pd.wait()


def kernel(xs0, xs1, xs2, W0, W1, W2, bos0, bos1, bos2, eos0, eos1, eos2):
    out = _encode(
        xs0.reshape(IDXROWS, TILE), xs1.reshape(IDXROWS, TILE),
        xs2.reshape(IDXROWS, TILE),
        W0, W1, W2,
        bos0.reshape(D), bos1.reshape(D), bos2.reshape(D),
        eos0.reshape(D), eos1.reshape(D), eos2.reshape(D),
    )
    return out.reshape(B, SEQ, DTOT)
